# Initial kernel scaffold; baseline (speedup 1.0000x reference)
import jax
import jax.numpy as jnp
from jax.experimental import pallas as pl


def kernel(x, edge_index, W1, b1, W2, b2, Wc1, bc1, Wc2, bc2, Wn1, bn1, Wn2, bn2):
    raise NotImplementedError("write your pallas kernel here")


# speedup vs baseline: 2.2758x; 2.2758x over previous
import functools

import jax
import jax.numpy as jnp
from jax import lax
from jax.experimental import pallas as pl
from jax.experimental.pallas import tpu as pltpu
from jax.experimental.pallas import tpu_sc as plsc

_N = 10000
_E = 320000
_NF = 128
_CF = 3
_TROW = 144
_SROW = 32
_MSG = 16

_NC = 2
_NS = 16
_NW = _NC * _NS
_EPW = _E // _NW
_CHUNK = 80
_ITERS = _EPW // _CHUNK
_NPS = _N // _NS

_sc_mesh = plsc.VectorSubcoreMesh(core_axis_name="c", subcore_axis_name="s")
_sc_params = pltpu.CompilerParams(use_tc_tiling_on_sc=False)


def _silu(v):
    return v * jax.nn.sigmoid(v)


@functools.partial(
    pl.kernel,
    out_type=[
        jax.ShapeDtypeStruct((_E, _TROW), jnp.float32),
        jax.ShapeDtypeStruct((_E, _TROW), jnp.float32),
    ],
    mesh=_sc_mesh,
    compiler_params=_sc_params,
    scratch_types=[
        pltpu.VMEM((_CHUNK,), jnp.int32),
        pltpu.VMEM((_CHUNK,), jnp.int32),
        pltpu.VMEM((_CHUNK, _TROW), jnp.float32),
        pltpu.VMEM((_CHUNK, _TROW), jnp.float32),
        pltpu.SemaphoreType.DMA,
    ],
)
def _k1_gather(t_hbm, dsti_hbm, srci_hbm, gd_hbm, gs_hbm,
               idxd_v, idxs_v, rowsd_v, rowss_v, sem):
    wid = lax.axis_index("s") * _NC + lax.axis_index("c")
    base0 = wid * _EPW

    def body(i, carry):
        base = base0 + i * _CHUNK
        pltpu.sync_copy(dsti_hbm.at[pl.ds(base, _CHUNK)], idxd_v)
        pltpu.sync_copy(srci_hbm.at[pl.ds(base, _CHUNK)], idxs_v)
        cp1 = pltpu.async_copy(t_hbm.at[idxd_v], rowsd_v, sem)
        cp2 = pltpu.async_copy(t_hbm.at[idxs_v], rowss_v, sem)
        cp1.wait()
        cp2.wait()
        pltpu.sync_copy(rowsd_v, gd_hbm.at[pl.ds(base, _CHUNK)])
        pltpu.sync_copy(rowss_v, gs_hbm.at[pl.ds(base, _CHUNK)])
        return carry

    lax.fori_loop(0, _ITERS, body, 0)


def _k2_body(gd, gs, W1, b1, W2, b2, Wc1, bc1, Wc2, bc2, out):
    xi = gd[:, :_NF]
    xj = gs[:, :_NF]
    rel = gs[:, _NF:_NF + _CF] - gd[:, _NF:_NF + _CF]
    dist = jnp.sqrt(jnp.sum(rel * rel, axis=1, keepdims=True))
    t = jnp.dot(xi, W1[0:_NF, :], preferred_element_type=jnp.float32)
    t = t + jnp.dot(xj, W1[_NF:2 * _NF, :], preferred_element_type=jnp.float32)
    t = t + dist * W1[2 * _NF:2 * _NF + 1, :]
    t = t + b1
    u = _silu(t)
    m = _silu(jnp.dot(u, W2, preferred_element_type=jnp.float32) + b2)
    cw = _silu(jnp.dot(m, Wc1, preferred_element_type=jnp.float32) + bc1)
    cw = jnp.dot(cw, Wc2, preferred_element_type=jnp.float32) + bc2
    pad = jnp.zeros((gd.shape[0], _SROW - _MSG - 1 - _CF), jnp.float32)
    out[:, :] = jnp.concatenate([m, cw, rel, pad], axis=1)


def _k2_edge_mlp(gd, gs, W1, b1, W2, b2, Wc1, bc1, Wc2, bc2, block):
    nblk = _E // block
    full = lambda i: (0, 0)
    return pl.pallas_call(
        lambda *refs: _k2_body(*[r[...] for r in refs[:-1]], refs[-1]),
        grid=(nblk,),
        in_specs=[
            pl.BlockSpec((block, _TROW), lambda i: (i, 0)),
            pl.BlockSpec((block, _TROW), lambda i: (i, 0)),
            pl.BlockSpec(W1.shape, full),
            pl.BlockSpec(b1.shape, full),
            pl.BlockSpec(W2.shape, full),
            pl.BlockSpec(b2.shape, full),
            pl.BlockSpec(Wc1.shape, full),
            pl.BlockSpec(bc1.shape, full),
            pl.BlockSpec(Wc2.shape, full),
            pl.BlockSpec(bc2.shape, full),
        ],
        out_specs=pl.BlockSpec((block, _SROW), lambda i: (i, 0)),
        out_shape=jax.ShapeDtypeStruct((_E, _SROW), jnp.float32),
    )(gd, gs, W1, b1, W2, b2, Wc1, bc1, Wc2, bc2)


@functools.partial(
    pl.kernel,
    out_type=jax.ShapeDtypeStruct((_NC, _N, _SROW), jnp.float32),
    mesh=_sc_mesh,
    compiler_params=_sc_params,
    scratch_types=[
        pltpu.VMEM((_CHUNK,), jnp.int32),
        pltpu.VMEM((_CHUNK, _SROW), jnp.float32),
        pltpu.VMEM((_NPS, _SROW), jnp.float32),
        pltpu.VMEM_SHARED((_N, _SROW), jnp.float32),
        pltpu.SemaphoreType.DMA,
    ],
)
def _k3_scatter(s_hbm, dsti_hbm, zer_hbm, out_hbm,
                idx_v, vals_v, stage_v, acc_sh, sem):
    cid = lax.axis_index("c")
    sid = lax.axis_index("s")
    wid = sid * _NC + cid
    base0 = wid * _EPW

    pltpu.sync_copy(zer_hbm.at[pl.ds(sid * _NPS, _NPS)], stage_v)
    pltpu.sync_copy(stage_v, acc_sh.at[pl.ds(sid * _NPS, _NPS)])
    plsc.subcore_barrier()

    def body(i, carry):
        base = base0 + i * _CHUNK
        pltpu.sync_copy(dsti_hbm.at[pl.ds(base, _CHUNK)], idx_v)
        pltpu.sync_copy(s_hbm.at[pl.ds(base, _CHUNK)], vals_v)
        pltpu.sync_copy(vals_v, acc_sh.at[idx_v], add=True)
        return carry

    lax.fori_loop(0, _ITERS, body, 0)
    plsc.subcore_barrier()

    pltpu.sync_copy(acc_sh.at[pl.ds(sid * _NPS, _NPS)], stage_v)
    pltpu.sync_copy(stage_v, out_hbm.at[cid, pl.ds(sid * _NPS, _NPS)])


def _k4_body(xb, pb, Wn1, bn1, Wn2, bn2, out):
    h = xb[:, :_NF]
    coords = xb[:, _NF:_NF + _CF]
    p = pb[0] + pb[1]
    m = p[:, :_MSG]
    cw = p[:, _MSG:_MSG + 1]
    cr = p[:, _MSG + 1:_MSG + 1 + _CF]
    coords_out = coords + cw * cr
    t = jnp.dot(h, Wn1[:_NF, :], preferred_element_type=jnp.float32)
    t = t + jnp.dot(m, Wn1[_NF:_NF + _MSG, :], preferred_element_type=jnp.float32)
    t = _silu(t + bn1)
    ho = jnp.dot(t, Wn2, preferred_element_type=jnp.float32) + bn2 + h
    out[:, :] = jnp.concatenate([ho, coords_out], axis=1)


def _k4_node_mlp(x, p, Wn1, bn1, Wn2, bn2, block):
    nblk = _N // block
    full = lambda i: (0, 0)
    return pl.pallas_call(
        lambda *refs: _k4_body(*[r[...] for r in refs[:-1]], refs[-1]),
        grid=(nblk,),
        in_specs=[
            pl.BlockSpec((block, _NF + _CF), lambda i: (i, 0)),
            pl.BlockSpec((_NC, block, _SROW), lambda i: (0, i, 0)),
            pl.BlockSpec(Wn1.shape, full),
            pl.BlockSpec(bn1.shape, full),
            pl.BlockSpec(Wn2.shape, full),
            pl.BlockSpec(bn2.shape, full),
        ],
        out_specs=pl.BlockSpec((block, _NF + _CF), lambda i: (i, 0)),
        out_shape=jax.ShapeDtypeStruct((_N, _NF + _CF), jnp.float32),
    )(x, p, Wn1, bn1, Wn2, bn2)


def kernel(x, edge_index, W1, b1, W2, b2, Wc1, bc1, Wc2, bc2, Wn1, bn1, Wn2, bn2):
    tpad = jnp.pad(x, ((0, 0), (0, _TROW - (_NF + _CF))))
    srci = edge_index[0]
    dsti = edge_index[1]

    gd, gs = _k1_gather(tpad, dsti, srci)
    s = _k2_edge_mlp(gd, gs, W1, b1.reshape(1, -1), W2, b2.reshape(1, -1),
                     Wc1, bc1.reshape(1, -1), Wc2, bc2.reshape(1, -1),
                     block=1000)
    zer = jnp.zeros((_N, _SROW), jnp.float32)
    p = _k3_scatter(s, dsti, zer)
    out = _k4_node_mlp(x, p, Wn1, bn1.reshape(1, -1), Wn2, bn2.reshape(1, -1),
                       block=2000)
    return out
